# baseline (speedup 1.0000x reference)
import functools

import jax
import jax.numpy as jnp
from jax import lax
from jax.experimental import pallas as pl
from jax.experimental.pallas import tpu as pltpu
from jax.experimental.pallas import tpu_sc as plsc

_N = 50000
_E = 800000
_D = 64
_H = 32
_NC = 2
_NS = 16
_NP = 51200
_RPT = _NP // _NS
_K = 128
_ROWS = 1024
_GRID = 49

@functools.cache
def _mesh():
    return plsc.VectorSubcoreMesh(core_axis_name="c", subcore_axis_name="s",
                                  num_cores=_NC, num_subcores=_NS)


def _deg_body(ei_h, odeg_h, ideg_h,
              is0, is1, is2, is3, id0, id1, id2, id3, ones_k, wb,
              sa0, sa1, sa2, sa3, sb0, sb1, sb2, sb3,
              so0, so1, so2, so3, si0, si1, si2, si3,
              odeg_sp, ideg_sp):
    c = lax.axis_index("c")
    s = lax.axis_index("s")

    z16 = jnp.zeros((16,), jnp.float32)
    for j in range(320 // 16):
        wb[pl.ds(j * 16, 16)] = z16
    row0 = s * _RPT

    def _zero(j, carry):
        pltpu.sync_copy(wb, odeg_sp.at[pl.ds(row0 + j * 320, 320)])
        pltpu.sync_copy(wb, ideg_sp.at[pl.ds(row0 + j * 320, 320)])
        return carry

    lax.fori_loop(0, _RPT // 320, _zero, 0)

    o16 = jnp.ones((16,), jnp.float32)
    for j in range(_K // 16):
        ones_k[pl.ds(j * 16, 16)] = o16

    plsc.subcore_barrier()

    nch = 195 + 5 * jnp.where(s == 15, 1, 0)
    ebase = c * (_E // _NC) + s * (195 * _K)

    slots = ((is0, id0, sa0, sb0, so0, si0), (is1, id1, sa1, sb1, so1, si1),
             (is2, id2, sa2, sb2, so2, si2), (is3, id3, sa3, sb3, so3, si3))
    S = len(slots)

    def issue_idx(i, slot):
        isr, idr, sma, smb, _, _ = slot
        off = ebase + i * _K
        pltpu.async_copy(ei_h.at[0, pl.ds(off, _K)], isr, sma)
        pltpu.async_copy(ei_h.at[1, pl.ds(off, _K)], idr, smb)

    for k in range(S):
        @pl.when(k < nch)
        def _(k=k):
            issue_idx(k, slots[k])

    def body(p, carry):
        i0 = p * S
        for k in range(S):
            isr, idr, sma, smb, smo, smi = slots[k]

            @pl.when(i0 + k < nch)
            def _(isr=isr, idr=idr, sma=sma, smb=smb, smo=smo, smi=smi):
                pltpu.make_async_copy(ei_h.at[0, pl.ds(0, _K)], isr, sma).wait()
                pltpu.make_async_copy(ei_h.at[1, pl.ds(0, _K)], idr, smb).wait()
                pltpu.async_copy(ones_k, odeg_sp.at[isr], smo, add=True)
                pltpu.async_copy(ones_k, ideg_sp.at[idr], smi, add=True)

        for k in range(S):
            isr, idr, sma, smb, smo, smi = slots[k]

            @pl.when(i0 + k + S < nch)
            def _(k=k, isr=isr, idr=idr, smo=smo, smi=smi):
                pltpu.make_async_copy(ones_k, odeg_sp.at[isr], smo).wait()
                pltpu.make_async_copy(ones_k, ideg_sp.at[idr], smi).wait()
                issue_idx(i0 + k + S, slots[k])
        return carry

    lax.fori_loop(0, (nch + S - 1) // S, body, 0)

    for k in range(S):
        isr, idr, sma, smb, smo, smi = slots[k]
        pltpu.make_async_copy(ones_k, odeg_sp.at[isr], smo).wait()
        pltpu.make_async_copy(ones_k, ideg_sp.at[idr], smi).wait()

    plsc.subcore_barrier()

    def wbl(j, carry):
        r = row0 + j * 320
        pltpu.sync_copy(odeg_sp.at[pl.ds(r, 320)], wb)
        pltpu.sync_copy(wb, odeg_h.at[c, pl.ds(r, 320)])
        pltpu.sync_copy(ideg_sp.at[pl.ds(r, 320)], wb)
        pltpu.sync_copy(wb, ideg_h.at[c, pl.ds(r, 320)])
        return carry

    lax.fori_loop(0, _RPT // 320, wbl, 0)


def _deg_call(edge_index):
    f = pl.kernel(
        _deg_body,
        out_type=(jax.ShapeDtypeStruct((_NC, _NP), jnp.float32),
                  jax.ShapeDtypeStruct((_NC, _NP), jnp.float32)),
        mesh=_mesh(),
        scratch_types=(
            [pltpu.VMEM((_K,), jnp.int32)] * 8
            + [pltpu.VMEM((_K,), jnp.float32), pltpu.VMEM((320,), jnp.float32)]
            + [pltpu.SemaphoreType.DMA] * 16
            + [pltpu.VMEM_SHARED((_NP,), jnp.float32),
               pltpu.VMEM_SHARED((_NP,), jnp.float32)]
        ),
        compiler_params=pltpu.CompilerParams(use_tc_tiling_on_sc=False),
    )
    return f(edge_index)


def _agg_body(feat_a, feat_b, ei_h, agg_h,
              is0, is1, is2, is3, id0, id1, id2, id3,
              r0, r1, r2, r3, wb,
              sa0, sa1, sa2, sa3, sb0, sb1, sb2, sb3,
              sg0, sg1, sg2, sg3, sc0, sc1, sc2, sc3,
              agg_sp):
    c = lax.axis_index("c")
    s = lax.axis_index("s")

    z16 = jnp.zeros((16,), jnp.float32)

    def _zwb(j, carry):
        wb[j, pl.ds(0, 16)] = z16
        wb[j, pl.ds(16, 16)] = z16
        return carry

    lax.fori_loop(0, 200, _zwb, 0)
    row0 = s * _RPT

    def _zero(j, carry):
        pltpu.sync_copy(wb, agg_sp.at[pl.ds(row0 + j * 200, 200)])
        return carry

    lax.fori_loop(0, _RPT // 200, _zero, 0)
    plsc.subcore_barrier()

    nch = 390 + 10 * jnp.where(s == 15, 1, 0)
    ebase = s * (390 * _K)

    slots = ((is0, id0, r0, sa0, sb0, sg0, sc0),
             (is1, id1, r1, sa1, sb1, sg1, sc1),
             (is2, id2, r2, sa2, sb2, sg2, sc2),
             (is3, id3, r3, sa3, sb3, sg3, sc3))
    S = len(slots)

    def issue_idx(i, slot):
        isr, idr, rw, sma, smb, smg, smc = slot
        off = ebase + i * _K
        pltpu.async_copy(ei_h.at[0, pl.ds(off, _K)], isr, sma)
        pltpu.async_copy(ei_h.at[1, pl.ds(off, _K)], idr, smb)

    def issue_gather(slot):
        isr, idr, rw, sma, smb, smg, smc = slot
        pltpu.make_async_copy(ei_h.at[0, pl.ds(0, _K)], isr, sma).wait()

        @pl.when(c == 0)
        def _():
            pltpu.async_copy(feat_a.at[isr], rw, smg)

        @pl.when(c == 1)
        def _():
            pltpu.async_copy(feat_b.at[isr], rw, smg)

    for k in range(S):
        issue_idx(k, slots[k])
    for k in range(S):
        issue_gather(slots[k])

    def body(p, carry):
        i0 = p * S
        for k in range(S):
            isr, idr, rw, sma, smb, smg, smc = slots[k]

            @pl.when(i0 + k < nch)
            def _(isr=isr, idr=idr, rw=rw, smb=smb, smg=smg, smc=smc):
                pltpu.make_async_copy(feat_a.at[isr], rw, smg).wait()
                pltpu.make_async_copy(ei_h.at[1, pl.ds(0, _K)], idr, smb).wait()
                pltpu.async_copy(rw, agg_sp.at[idr], smc, add=True)

        for k in range(S):
            isr, idr, rw, sma, smb, smg, smc = slots[k]

            @pl.when(i0 + k + S < nch)
            def _(k=k, isr=isr, idr=idr, rw=rw, smc=smc):
                pltpu.make_async_copy(rw, agg_sp.at[idr], smc).wait()
                issue_idx(i0 + k + S, slots[k])

        for k in range(S):
            @pl.when(i0 + k + S < nch)
            def _(k=k):
                issue_gather(slots[k])
        return carry

    lax.fori_loop(0, (nch + S - 1) // S, body, 0)

    for k in range(S):
        isr, idr, rw, sma, smb, smg, smc = slots[k]
        pltpu.make_async_copy(rw, agg_sp.at[idr], smc).wait()

    plsc.subcore_barrier()

    def wbl(j, carry):
        r = row0 + j * 200
        pltpu.sync_copy(agg_sp.at[pl.ds(r, 200)], wb)
        pltpu.sync_copy(wb, agg_h.at[c, pl.ds(r, 200)])
        return carry

    lax.fori_loop(0, _RPT // 200, wbl, 0)


def _agg_call(feat_a, feat_b, edge_index):
    f = pl.kernel(
        _agg_body,
        out_type=jax.ShapeDtypeStruct((_NC, _NP, _H), jnp.float32),
        mesh=_mesh(),
        scratch_types=(
            [pltpu.VMEM((_K,), jnp.int32)] * 8
            + [pltpu.VMEM((_K, _H), jnp.float32)] * 4
            + [pltpu.VMEM((200, _H), jnp.float32)]
            + [pltpu.SemaphoreType.DMA] * 16
            + [pltpu.VMEM_SHARED((_NP, _H), jnp.float32)]
        ),
        compiler_params=pltpu.CompilerParams(use_tc_tiling_on_sc=False),
    )
    return f(feat_a, feat_b, edge_index)


def _feat_body(x_ref, d_ref, oa_ref, ob_ref):
    d2 = d_ref[...]
    od = d2[:, 0:1] + d2[:, 1:2]
    norm = lax.rsqrt(jnp.maximum(od, 1.0))
    f = x_ref[...] * norm
    oa_ref[...] = f[:, :_H]
    ob_ref[...] = f[:, _H:]


def _feat_call(x, odeg2):
    return pl.pallas_call(
        _feat_body,
        grid=(_GRID,),
        in_specs=[
            pl.BlockSpec((_ROWS, _D), lambda i: (i, 0)),
            pl.BlockSpec((_ROWS, _NC), lambda i: (i, 0)),
        ],
        out_specs=[
            pl.BlockSpec((_ROWS, _H), lambda i: (i, 0)),
            pl.BlockSpec((_ROWS, _H), lambda i: (i, 0)),
        ],
        out_shape=[jax.ShapeDtypeStruct((_NP, _H), jnp.float32),
                   jax.ShapeDtypeStruct((_NP, _H), jnp.float32)],
    )(x, odeg2)


def _final_body(a_ref, d_ref, wgc_ref, bgc_ref, w2_ref, b2_ref, o_ref):
    a = a_ref[...]
    rst = jnp.concatenate([a[0], a[1]], axis=1)
    d2 = d_ref[...]
    norm = lax.rsqrt(jnp.maximum(d2[:, 0:1] + d2[:, 1:2], 1.0))
    rst = rst * norm
    h = jnp.maximum(
        jnp.dot(rst, wgc_ref[...], preferred_element_type=jnp.float32)
        + bgc_ref[...], 0.0)
    o_ref[...] = (
        jnp.dot(h, w2_ref[...], preferred_element_type=jnp.float32)
        + b2_ref[...])


def _final_call(agg2, ideg2, gc_weight, gc_bias, w2, b2):
    return pl.pallas_call(
        _final_body,
        grid=(_GRID,),
        in_specs=[
            pl.BlockSpec((_NC, _ROWS, _H), lambda i: (0, i, 0)),
            pl.BlockSpec((_ROWS, _NC), lambda i: (i, 0)),
            pl.BlockSpec((_D, _D), lambda i: (0, 0)),
            pl.BlockSpec((1, _D), lambda i: (0, 0)),
            pl.BlockSpec((_D, _D), lambda i: (0, 0)),
            pl.BlockSpec((1, _D), lambda i: (0, 0)),
        ],
        out_specs=pl.BlockSpec((_ROWS, _D), lambda i: (i, 0)),
        out_shape=jax.ShapeDtypeStruct((_N, _D), jnp.float32),
    )(agg2, ideg2, gc_weight, gc_bias, w2, b2)


def kernel(node_embeddings, gc_weight, gc_bias, in_proj_weight, in_proj_bias,
           out_proj_weight, out_proj_bias, edge_index):
    odeg2, ideg2 = _deg_call(edge_index)
    feat_a, feat_b = _feat_call(node_embeddings, odeg2.T)
    agg3 = _agg_call(feat_a, feat_b, edge_index)
    w2 = in_proj_weight[2 * _D:].T @ out_proj_weight.T
    b2 = (in_proj_bias[2 * _D:] @ out_proj_weight.T + out_proj_bias)[None, :]
    return _final_call(agg3, ideg2.T, gc_weight, gc_bias[None, :], w2, b2)


# speedup vs baseline: 1.1923x; 1.1923x over previous
import functools

import jax
import jax.numpy as jnp
from jax import lax
from jax.experimental import pallas as pl
from jax.experimental.pallas import tpu as pltpu
from jax.experimental.pallas import tpu_sc as plsc

_N = 50000
_E = 800000
_D = 64
_H = 32
_NC = 2
_NS = 16
_NP = 51200
_RPT = _NP // _NS
_K = 128
_ROWS = 1024
_GRID = 49

@functools.cache
def _mesh():
    return plsc.VectorSubcoreMesh(core_axis_name="c", subcore_axis_name="s",
                                  num_cores=_NC, num_subcores=_NS)


def _deg_body(ei_h, odeg_h, ideg_h,
              is0, is1, is2, is3, id0, id1, id2, id3, ones_k, wb,
              sa0, sa1, sa2, sa3, sb0, sb1, sb2, sb3,
              so0, so1, so2, so3, si0, si1, si2, si3,
              odeg_sp, ideg_sp):
    c = lax.axis_index("c")
    s = lax.axis_index("s")

    z16 = jnp.zeros((16,), jnp.float32)
    for j in range(320 // 16):
        wb[pl.ds(j * 16, 16)] = z16
    row0 = s * _RPT

    def _zero(j, carry):
        pltpu.sync_copy(wb, odeg_sp.at[pl.ds(row0 + j * 320, 320)])
        pltpu.sync_copy(wb, ideg_sp.at[pl.ds(row0 + j * 320, 320)])
        return carry

    lax.fori_loop(0, _RPT // 320, _zero, 0)

    o16 = jnp.ones((16,), jnp.float32)
    for j in range(_K // 16):
        ones_k[pl.ds(j * 16, 16)] = o16

    plsc.subcore_barrier()

    nch = 195 + 5 * jnp.where(s == 15, 1, 0)
    ebase = c * (_E // _NC) + s * (195 * _K)

    slots = ((is0, id0, sa0, sb0, so0, si0), (is1, id1, sa1, sb1, so1, si1),
             (is2, id2, sa2, sb2, so2, si2), (is3, id3, sa3, sb3, so3, si3))
    S = len(slots)

    def issue_idx(i, slot):
        isr, idr, sma, smb, _, _ = slot
        off = ebase + i * _K
        pltpu.async_copy(ei_h.at[0, pl.ds(off, _K)], isr, sma)
        pltpu.async_copy(ei_h.at[1, pl.ds(off, _K)], idr, smb)

    for k in range(S):
        @pl.when(k < nch)
        def _(k=k):
            issue_idx(k, slots[k])

    def body(p, carry):
        i0 = p * S
        for k in range(S):
            isr, idr, sma, smb, smo, smi = slots[k]

            @pl.when(i0 + k < nch)
            def _(isr=isr, idr=idr, sma=sma, smb=smb, smo=smo, smi=smi):
                pltpu.make_async_copy(ei_h.at[0, pl.ds(0, _K)], isr, sma).wait()
                pltpu.make_async_copy(ei_h.at[1, pl.ds(0, _K)], idr, smb).wait()
                pltpu.async_copy(ones_k, odeg_sp.at[isr], smo, add=True)
                pltpu.async_copy(ones_k, ideg_sp.at[idr], smi, add=True)

        for k in range(S):
            isr, idr, sma, smb, smo, smi = slots[k]

            @pl.when(i0 + k + S < nch)
            def _(k=k, isr=isr, idr=idr, smo=smo, smi=smi):
                pltpu.make_async_copy(ones_k, odeg_sp.at[isr], smo).wait()
                pltpu.make_async_copy(ones_k, ideg_sp.at[idr], smi).wait()
                issue_idx(i0 + k + S, slots[k])
        return carry

    lax.fori_loop(0, (nch + S - 1) // S, body, 0)

    for k in range(S):
        isr, idr, sma, smb, smo, smi = slots[k]
        pltpu.make_async_copy(ones_k, odeg_sp.at[isr], smo).wait()
        pltpu.make_async_copy(ones_k, ideg_sp.at[idr], smi).wait()

    plsc.subcore_barrier()

    def wbl(j, carry):
        r = row0 + j * 320
        pltpu.sync_copy(odeg_sp.at[pl.ds(r, 320)], wb)
        pltpu.sync_copy(wb, odeg_h.at[c, pl.ds(r, 320)])
        pltpu.sync_copy(ideg_sp.at[pl.ds(r, 320)], wb)
        pltpu.sync_copy(wb, ideg_h.at[c, pl.ds(r, 320)])
        return carry

    lax.fori_loop(0, _RPT // 320, wbl, 0)


def _deg_call(edge_index):
    f = pl.kernel(
        _deg_body,
        out_type=(jax.ShapeDtypeStruct((_NC, _NP), jnp.float32),
                  jax.ShapeDtypeStruct((_NC, _NP), jnp.float32)),
        mesh=_mesh(),
        scratch_types=(
            [pltpu.VMEM((_K,), jnp.int32)] * 8
            + [pltpu.VMEM((_K,), jnp.float32), pltpu.VMEM((320,), jnp.float32)]
            + [pltpu.SemaphoreType.DMA] * 16
            + [pltpu.VMEM_SHARED((_NP,), jnp.float32),
               pltpu.VMEM_SHARED((_NP,), jnp.float32)]
        ),
        compiler_params=pltpu.CompilerParams(use_tc_tiling_on_sc=False),
    )
    return f(edge_index)


def _agg_body(feat_i, ei_h, agg_h,
              is0, is1, is2, is3, id0, id1, id2, id3,
              r0, r1, r2, r3, wb,
              sa0, sa1, sa2, sa3, sb0, sb1, sb2, sb3,
              sg0, sg1, sg2, sg3, sc0, sc1, sc2, sc3,
              agg_sp):
    c = lax.axis_index("c")
    s = lax.axis_index("s")

    z16 = jnp.zeros((16,), jnp.float32)

    def _zwb(j, carry):
        wb[j, pl.ds(0, 16)] = z16
        wb[j, pl.ds(16, 16)] = z16
        return carry

    lax.fori_loop(0, 200, _zwb, 0)
    row0 = s * _RPT

    def _zero(j, carry):
        pltpu.sync_copy(wb, agg_sp.at[pl.ds(row0 + j * 200, 200)])
        return carry

    lax.fori_loop(0, _RPT // 200, _zero, 0)
    plsc.subcore_barrier()

    nch = 390 + 10 * jnp.where(s == 15, 1, 0)
    ebase = s * (390 * _K)

    slots = ((is0, id0, r0, sa0, sb0, sg0, sc0),
             (is1, id1, r1, sa1, sb1, sg1, sc1),
             (is2, id2, r2, sa2, sb2, sg2, sc2),
             (is3, id3, r3, sa3, sb3, sg3, sc3))
    S = len(slots)

    def issue_idx(i, slot):
        isr, idr, rw, sma, smb, smg, smc = slot
        off = ebase + i * _K
        pltpu.async_copy(ei_h.at[0, pl.ds(off, _K)], isr, sma)
        pltpu.async_copy(ei_h.at[1, pl.ds(off, _K)], idr, smb)

    def issue_gather(slot):
        isr, idr, rw, sma, smb, smg, smc = slot
        pltpu.make_async_copy(ei_h.at[0, pl.ds(0, _K)], isr, sma).wait()
        for j in range(_K // 16):
            v = isr[pl.ds(j * 16, 16)]
            isr[pl.ds(j * 16, 16)] = v * 2 + c
        pltpu.async_copy(feat_i.at[isr], rw, smg)

    for k in range(S):
        issue_idx(k, slots[k])
    for k in range(S):
        issue_gather(slots[k])

    def body(p, carry):
        i0 = p * S
        for k in range(S):
            isr, idr, rw, sma, smb, smg, smc = slots[k]

            @pl.when(i0 + k < nch)
            def _(isr=isr, idr=idr, rw=rw, smb=smb, smg=smg, smc=smc):
                pltpu.make_async_copy(feat_i.at[isr], rw, smg).wait()
                pltpu.make_async_copy(ei_h.at[1, pl.ds(0, _K)], idr, smb).wait()
                pltpu.async_copy(rw, agg_sp.at[idr], smc, add=True)

        for k in range(S):
            isr, idr, rw, sma, smb, smg, smc = slots[k]

            @pl.when(i0 + k + S < nch)
            def _(k=k, isr=isr, idr=idr, rw=rw, smc=smc):
                pltpu.make_async_copy(rw, agg_sp.at[idr], smc).wait()
                issue_idx(i0 + k + S, slots[k])

        for k in range(S):
            @pl.when(i0 + k + S < nch)
            def _(k=k):
                issue_gather(slots[k])
        return carry

    lax.fori_loop(0, (nch + S - 1) // S, body, 0)

    for k in range(S):
        isr, idr, rw, sma, smb, smg, smc = slots[k]
        pltpu.make_async_copy(rw, agg_sp.at[idr], smc).wait()

    plsc.subcore_barrier()

    def wbl(j, carry):
        r = row0 + j * 200
        pltpu.sync_copy(agg_sp.at[pl.ds(r, 200)], wb)
        pltpu.sync_copy(wb, agg_h.at[pl.ds(r, 200), pl.ds(c * _H, _H)])
        return carry

    lax.fori_loop(0, _RPT // 200, wbl, 0)


def _agg_call(feat_i, edge_index):
    f = pl.kernel(
        _agg_body,
        out_type=jax.ShapeDtypeStruct((_NP, _D), jnp.float32),
        mesh=_mesh(),
        scratch_types=(
            [pltpu.VMEM((_K,), jnp.int32)] * 8
            + [pltpu.VMEM((_K, _H), jnp.float32)] * 4
            + [pltpu.VMEM((200, _H), jnp.float32)]
            + [pltpu.SemaphoreType.DMA] * 16
            + [pltpu.VMEM_SHARED((_NP, _H), jnp.float32)]
        ),
        compiler_params=pltpu.CompilerParams(use_tc_tiling_on_sc=False),
    )
    return f(feat_i, edge_index)


def _feat_body(x_ref, d_ref, o_ref):
    d2 = d_ref[...]
    od = d2[0, :] + d2[1, :]
    norm = lax.rsqrt(jnp.maximum(od, 1.0))[:, None]
    o_ref[...] = x_ref[...] * norm


def _feat_call(x, odeg2):
    return pl.pallas_call(
        _feat_body,
        grid=(_GRID,),
        in_specs=[
            pl.BlockSpec((_ROWS, _D), lambda i: (i, 0)),
            pl.BlockSpec((_NC, _ROWS), lambda i: (0, i)),
        ],
        out_specs=pl.BlockSpec((_ROWS, _D), lambda i: (i, 0)),
        out_shape=jax.ShapeDtypeStruct((_NP, _D), jnp.float32),
    )(x, odeg2)


def _final_body(a_ref, d_ref, wgc_ref, bgc_ref, w2_ref, b2_ref, o_ref):
    d2 = d_ref[...]
    norm = lax.rsqrt(jnp.maximum(d2[0, :] + d2[1, :], 1.0))[:, None]
    rst = a_ref[...] * norm
    h = jnp.maximum(
        jnp.dot(rst, wgc_ref[...], preferred_element_type=jnp.float32)
        + bgc_ref[...], 0.0)
    o_ref[...] = (
        jnp.dot(h, w2_ref[...], preferred_element_type=jnp.float32)
        + b2_ref[...])


def _final_call(agg2, ideg2, gc_weight, gc_bias, w2, b2):
    return pl.pallas_call(
        _final_body,
        grid=(_GRID,),
        in_specs=[
            pl.BlockSpec((_ROWS, _D), lambda i: (i, 0)),
            pl.BlockSpec((_NC, _ROWS), lambda i: (0, i)),
            pl.BlockSpec((_D, _D), lambda i: (0, 0)),
            pl.BlockSpec((1, _D), lambda i: (0, 0)),
            pl.BlockSpec((_D, _D), lambda i: (0, 0)),
            pl.BlockSpec((1, _D), lambda i: (0, 0)),
        ],
        out_specs=pl.BlockSpec((_ROWS, _D), lambda i: (i, 0)),
        out_shape=jax.ShapeDtypeStruct((_N, _D), jnp.float32),
    )(agg2, ideg2, gc_weight, gc_bias, w2, b2)


def kernel(node_embeddings, gc_weight, gc_bias, in_proj_weight, in_proj_bias,
           out_proj_weight, out_proj_bias, edge_index):
    odeg2, ideg2 = _deg_call(edge_index)
    feat64 = _feat_call(node_embeddings, odeg2)
    agg3 = _agg_call(feat64.reshape(2 * _NP, _H), edge_index)
    w2 = in_proj_weight[2 * _D:].T @ out_proj_weight.T
    b2 = (in_proj_bias[2 * _D:] @ out_proj_weight.T + out_proj_bias)[None, :]
    return _final_call(agg3, ideg2, gc_weight, gc_bias[None, :], w2, b2)


# speedup vs baseline: 1.2884x; 1.0806x over previous
import functools

import jax
import jax.numpy as jnp
from jax import lax
from jax.experimental import pallas as pl
from jax.experimental.pallas import tpu as pltpu
from jax.experimental.pallas import tpu_sc as plsc

_N = 50000
_E = 800000
_D = 64
_H = 32
_NC = 2
_NS = 16
_NP = 51200
_RPT = _NP // _NS
_K = 128
_ROWS = 2048
_GRID = 25

@functools.cache
def _mesh():
    return plsc.VectorSubcoreMesh(core_axis_name="c", subcore_axis_name="s",
                                  num_cores=_NC, num_subcores=_NS)


def _make_deg_body(row):
    def body(ei_h, deg_h,
             i0, i1, i2, i3, ones_k, wb,
             sa0, sa1, sa2, sa3, so0, so1, so2, so3,
             deg_sp):
        c = lax.axis_index("c")
        s = lax.axis_index("s")

        z16 = jnp.zeros((16,), jnp.float32)
        for j in range(320 // 16):
            wb[pl.ds(j * 16, 16)] = z16
        row0 = s * _RPT

        def _zero(j, carry):
            pltpu.sync_copy(wb, deg_sp.at[pl.ds(row0 + j * 320, 320)])
            return carry

        lax.fori_loop(0, _RPT // 320, _zero, 0)

        o16 = jnp.ones((16,), jnp.float32)
        for j in range(_K // 16):
            ones_k[pl.ds(j * 16, 16)] = o16

        plsc.subcore_barrier()

        nch = 195 + 5 * jnp.where(s == 15, 1, 0)
        ebase = c * (_E // _NC) + s * (195 * _K)

        slots = ((i0, sa0, so0), (i1, sa1, so1), (i2, sa2, so2), (i3, sa3, so3))
        S = len(slots)

        def issue_idx(i, slot):
            isr, sma, _ = slot
            off = ebase + i * _K
            pltpu.async_copy(ei_h.at[row, pl.ds(off, _K)], isr, sma)

        for k in range(S):
            issue_idx(k, slots[k])

        def lbody(p, carry):
            i0_ = p * S
            for k in range(S):
                isr, sma, smo = slots[k]

                @pl.when(i0_ + k < nch)
                def _(isr=isr, sma=sma, smo=smo):
                    pltpu.make_async_copy(
                        ei_h.at[row, pl.ds(0, _K)], isr, sma).wait()
                    pltpu.async_copy(ones_k, deg_sp.at[isr], smo, add=True)

            for k in range(S):
                isr, sma, smo = slots[k]

                @pl.when(i0_ + k + S < nch)
                def _(k=k, isr=isr, smo=smo):
                    pltpu.make_async_copy(ones_k, deg_sp.at[isr], smo).wait()
                    issue_idx(i0_ + k + S, slots[k])
            return carry

        lax.fori_loop(0, (nch + S - 1) // S, lbody, 0)

        for k in range(S):
            isr, sma, smo = slots[k]
            pltpu.make_async_copy(ones_k, deg_sp.at[isr], smo).wait()

        plsc.subcore_barrier()

        def wbl(j, carry):
            r = row0 + j * 320
            pltpu.sync_copy(deg_sp.at[pl.ds(r, 320)], wb)
            pltpu.sync_copy(wb, deg_h.at[c, pl.ds(r, 320)])
            return carry

        lax.fori_loop(0, _RPT // 320, wbl, 0)

    return body


def _deg_call(edge_index, row):
    f = pl.kernel(
        _make_deg_body(row),
        out_type=jax.ShapeDtypeStruct((_NC, _NP), jnp.float32),
        mesh=_mesh(),
        scratch_types=(
            [pltpu.VMEM((_K,), jnp.int32)] * 4
            + [pltpu.VMEM((_K,), jnp.float32), pltpu.VMEM((320,), jnp.float32)]
            + [pltpu.SemaphoreType.DMA] * 8
            + [pltpu.VMEM_SHARED((_NP,), jnp.float32)]
        ),
        compiler_params=pltpu.CompilerParams(use_tc_tiling_on_sc=False),
    )
    return f(edge_index)


def _agg_body(feat_i, ei_h, agg_h,
              is0, is1, is2, is3, id0, id1, id2, id3,
              r0, r1, r2, r3, wb,
              sa0, sa1, sa2, sa3, sb0, sb1, sb2, sb3,
              sg0, sg1, sg2, sg3, sc0, sc1, sc2, sc3,
              agg_sp):
    c = lax.axis_index("c")
    s = lax.axis_index("s")

    z16 = jnp.zeros((16,), jnp.float32)

    def _zwb(j, carry):
        wb[j, pl.ds(0, 16)] = z16
        wb[j, pl.ds(16, 16)] = z16
        return carry

    lax.fori_loop(0, 200, _zwb, 0)
    row0 = s * _RPT

    def _zero(j, carry):
        pltpu.sync_copy(wb, agg_sp.at[pl.ds(row0 + j * 200, 200)])
        return carry

    lax.fori_loop(0, _RPT // 200, _zero, 0)
    plsc.subcore_barrier()

    nch = 390 + 10 * jnp.where(s == 15, 1, 0)
    ebase = s * (390 * _K)

    slots = ((is0, id0, r0, sa0, sb0, sg0, sc0),
             (is1, id1, r1, sa1, sb1, sg1, sc1),
             (is2, id2, r2, sa2, sb2, sg2, sc2),
             (is3, id3, r3, sa3, sb3, sg3, sc3))
    S = len(slots)

    def issue_idx(i, slot):
        isr, idr, rw, sma, smb, smg, smc = slot
        off = ebase + i * _K
        pltpu.async_copy(ei_h.at[0, pl.ds(off, _K)], isr, sma)
        pltpu.async_copy(ei_h.at[1, pl.ds(off, _K)], idr, smb)

    def issue_gather(slot):
        isr, idr, rw, sma, smb, smg, smc = slot
        pltpu.make_async_copy(ei_h.at[0, pl.ds(0, _K)], isr, sma).wait()
        for j in range(_K // 16):
            v = isr[pl.ds(j * 16, 16)]
            isr[pl.ds(j * 16, 16)] = v * 2 + c
        pltpu.async_copy(feat_i.at[isr], rw, smg)

    for k in range(S):
        issue_idx(k, slots[k])
    for k in range(S):
        issue_gather(slots[k])

    def body(p, carry):
        i0 = p * S
        for k in range(S):
            isr, idr, rw, sma, smb, smg, smc = slots[k]

            @pl.when(i0 + k < nch)
            def _(isr=isr, idr=idr, rw=rw, smb=smb, smg=smg, smc=smc):
                pltpu.make_async_copy(feat_i.at[isr], rw, smg).wait()
                pltpu.make_async_copy(ei_h.at[1, pl.ds(0, _K)], idr, smb).wait()
                pltpu.async_copy(rw, agg_sp.at[idr], smc, add=True)

        for k in range(S):
            isr, idr, rw, sma, smb, smg, smc = slots[k]

            @pl.when(i0 + k + S < nch)
            def _(k=k, isr=isr, idr=idr, rw=rw, smc=smc):
                pltpu.make_async_copy(rw, agg_sp.at[idr], smc).wait()
                issue_idx(i0 + k + S, slots[k])

        for k in range(S):
            @pl.when(i0 + k + S < nch)
            def _(k=k):
                issue_gather(slots[k])
        return carry

    lax.fori_loop(0, (nch + S - 1) // S, body, 0)

    for k in range(S):
        isr, idr, rw, sma, smb, smg, smc = slots[k]
        pltpu.make_async_copy(rw, agg_sp.at[idr], smc).wait()

    plsc.subcore_barrier()

    def wbl(j, carry):
        r = row0 + j * 200
        pltpu.sync_copy(agg_sp.at[pl.ds(r, 200)], wb)
        pltpu.sync_copy(wb, agg_h.at[pl.ds(r, 200), pl.ds(c * _H, _H)])
        return carry

    lax.fori_loop(0, _RPT // 200, wbl, 0)


def _agg_call(feat_i, edge_index):
    f = pl.kernel(
        _agg_body,
        out_type=jax.ShapeDtypeStruct((_NP, _D), jnp.float32),
        mesh=_mesh(),
        scratch_types=(
            [pltpu.VMEM((_K,), jnp.int32)] * 8
            + [pltpu.VMEM((_K, _H), jnp.float32)] * 4
            + [pltpu.VMEM((200, _H), jnp.float32)]
            + [pltpu.SemaphoreType.DMA] * 16
            + [pltpu.VMEM_SHARED((_NP, _H), jnp.float32)]
        ),
        compiler_params=pltpu.CompilerParams(use_tc_tiling_on_sc=False),
    )
    return f(feat_i, edge_index)


def _feat_body(x_ref, d_ref, o_ref):
    d2 = d_ref[...]
    od = d2[0, :] + d2[1, :]
    norm = lax.rsqrt(jnp.maximum(od, 1.0))[:, None]
    o_ref[...] = x_ref[...] * norm


def _feat_call(x, odeg2):
    return pl.pallas_call(
        _feat_body,
        grid=(_GRID,),
        in_specs=[
            pl.BlockSpec((_ROWS, _D), lambda i: (i, 0)),
            pl.BlockSpec((_NC, _ROWS), lambda i: (0, i)),
        ],
        out_specs=pl.BlockSpec((_ROWS, _D), lambda i: (i, 0)),
        out_shape=jax.ShapeDtypeStruct((_NP, _D), jnp.float32),
    )(x, odeg2)


def _final_body(a_ref, d_ref, wgc_ref, bgc_ref, w2_ref, b2_ref, o_ref):
    d2 = d_ref[...]
    norm = lax.rsqrt(jnp.maximum(d2[0, :] + d2[1, :], 1.0))[:, None]
    rst = a_ref[...] * norm
    h = jnp.maximum(
        jnp.dot(rst, wgc_ref[...], preferred_element_type=jnp.float32)
        + bgc_ref[...], 0.0)
    o_ref[...] = (
        jnp.dot(h, w2_ref[...], preferred_element_type=jnp.float32)
        + b2_ref[...])


def _final_call(agg2, ideg2, gc_weight, gc_bias, w2, b2):
    return pl.pallas_call(
        _final_body,
        grid=(_GRID,),
        in_specs=[
            pl.BlockSpec((_ROWS, _D), lambda i: (i, 0)),
            pl.BlockSpec((_NC, _ROWS), lambda i: (0, i)),
            pl.BlockSpec((_D, _D), lambda i: (0, 0)),
            pl.BlockSpec((1, _D), lambda i: (0, 0)),
            pl.BlockSpec((_D, _D), lambda i: (0, 0)),
            pl.BlockSpec((1, _D), lambda i: (0, 0)),
        ],
        out_specs=pl.BlockSpec((_ROWS, _D), lambda i: (i, 0)),
        out_shape=jax.ShapeDtypeStruct((_N, _D), jnp.float32),
    )(agg2, ideg2, gc_weight, gc_bias, w2, b2)


def kernel(node_embeddings, gc_weight, gc_bias, in_proj_weight, in_proj_bias,
           out_proj_weight, out_proj_bias, edge_index):
    odeg2 = _deg_call(edge_index, 0)
    feat64 = _feat_call(node_embeddings, odeg2)
    ideg2 = _deg_call(edge_index, 1)
    agg3 = _agg_call(feat64.reshape(2 * _NP, _H), edge_index)
    w2 = in_proj_weight[2 * _D:].T @ out_proj_weight.T
    b2 = (in_proj_bias[2 * _D:] @ out_proj_weight.T + out_proj_bias)[None, :]
    return _final_call(agg3, ideg2, gc_weight, gc_bias[None, :], w2, b2)
